# Initial kernel scaffold; baseline (speedup 1.0000x reference)
import jax
import jax.numpy as jnp
from jax.experimental import pallas as pl


def kernel(x, edge_index, edge_weight, h_0, c_0, params):
    raise NotImplementedError("write your pallas kernel here")


# speedup vs baseline: 5.2697x; 5.2697x over previous
import functools

import jax
import jax.numpy as jnp
from jax import lax
from jax.experimental import pallas as pl
from jax.experimental.pallas import tpu as pltpu
from jax.experimental.pallas import tpu_sc as plsc

NC = 2
NS = 16
CHUNK = 128
ROWS_PER_STEP = 8
GCOLS = 16


def _make_agg_kernel(n_nodes, idx_rows, acc_rows):
    rows_per_sub = idx_rows // NS
    steps = rows_per_sub // ROWS_PER_STEP
    half_rows_per_sub = idx_rows // 2 // NS
    half_steps = half_rows_per_sub // ROWS_PER_STEP
    zrows = acc_rows // NS
    mesh = plsc.VectorSubcoreMesh(core_axis_name="c", subcore_axis_name="s")
    oshape = jax.ShapeDtypeStruct((acc_rows, GCOLS), jnp.float32)

    @functools.partial(
        pl.kernel,
        out_type=(oshape, oshape, oshape, oshape),
        mesh=mesh,
        scratch_types=[
            pltpu.VMEM((CHUNK,), jnp.int32),
            pltpu.VMEM((CHUNK,), jnp.int32),
            pltpu.VMEM((CHUNK, GCOLS), jnp.float32),
            pltpu.VMEM_SHARED((acc_rows, GCOLS), jnp.float32),
            pltpu.SemaphoreType.DMA,
        ],
        compiler_params=pltpu.CompilerParams(use_tc_tiling_on_sc=False),
    )
    def agg_kernel(t0_hbm, t1_hbm, t2_hbm, src_hbm, dst_hbm, zeros_hbm,
                   out0, out1, out2a, out2b,
                   idx1s, idx1d, rb, acc, sem):
        cid = lax.axis_index("c")
        sid = lax.axis_index("s")

        def zero_acc():
            pltpu.sync_copy(zeros_hbm.at[pl.ds(sid * zrows, zrows)],
                            acc.at[pl.ds(sid * zrows, zrows)])

        def scan(table, row_base, nsteps):
            def step(b, carry):
                r0 = row_base + b * ROWS_PER_STEP
                for j in range(ROWS_PER_STEP):
                    pltpu.sync_copy(src_hbm.at[r0 + j], idx1s)
                    pltpu.async_copy(table.at[idx1s], rb, sem).wait()
                    pltpu.sync_copy(dst_hbm.at[r0 + j], idx1d)
                    pltpu.sync_copy(rb, acc.at[idx1d], add=True)
                return carry
            lax.fori_loop(0, nsteps, step, 0)

        def dump(out):
            pltpu.sync_copy(acc.at[pl.ds(sid * zrows, zrows)],
                            out.at[pl.ds(sid * zrows, zrows)])

        zero_acc()
        plsc.subcore_barrier()

        @pl.when(cid == 0)
        def _():
            scan(t0_hbm, sid * rows_per_sub, steps)

        @pl.when(cid == 1)
        def _():
            scan(t1_hbm, sid * rows_per_sub, steps)
        plsc.subcore_barrier()

        @pl.when(cid == 0)
        def _():
            dump(out0)

        @pl.when(cid == 1)
        def _():
            dump(out1)
        plsc.subcore_barrier()

        zero_acc()
        plsc.subcore_barrier()
        scan(t2_hbm, cid * (idx_rows // 2) + sid * half_rows_per_sub,
             half_steps)
        plsc.subcore_barrier()

        @pl.when(cid == 0)
        def _():
            dump(out2a)

        @pl.when(cid == 1)
        def _():
            dump(out2b)

    return agg_kernel


def _dense_body(x_ref, h_ref, c0_ref, a0_ref, a1_ref, a2a_ref, a2b_ref,
                wx_ref, wh_ref, w0_ref, w1_ref, w2_ref, b_ref, lw_ref, lb_ref,
                out_ref, hn_ref, c_ref, *, filt, lags):
    def dot(a, b):
        return lax.dot_general(a, b, (((1,), (0,)), ((), ())),
                               precision=lax.Precision.HIGHEST,
                               preferred_element_type=jnp.float32)
    a0 = a0_ref[...]
    a1 = a1_ref[...]
    a2 = a2a_ref[...] + a2b_ref[...]
    inv = 1.0 / jnp.maximum(a2[:, lags:lags + 1], 1.0)
    pre = (dot(x_ref[...], wx_ref[...]) + dot(h_ref[...], wh_ref[...])
           + dot(a0 * inv, w0_ref[...]) + dot(a1 * inv, w1_ref[...])
           + dot(a2 * inv, w2_ref[...]) + b_ref[...])
    i_g = jax.nn.sigmoid(pre[:, 0 * filt:1 * filt])
    f_g = jax.nn.sigmoid(pre[:, 1 * filt:2 * filt])
    t_g = jnp.tanh(pre[:, 2 * filt:3 * filt])
    o_g = jax.nn.sigmoid(pre[:, 3 * filt:4 * filt])
    c = f_g * c0_ref[...] + i_g * t_g
    hn = o_g * jnp.tanh(c)
    h = jnp.maximum(hn, 0.0)
    out_ref[...] = dot(h, lw_ref[...]) + lb_ref[...]
    hn_ref[...] = hn
    c_ref[...] = c


def _make_dense_kernel(n_nodes, lags, filt, row_block):
    grid = (n_nodes // row_block,)
    g4 = 4 * filt
    row = lambda i: (i, 0)
    rep = lambda i: (0, 0)
    return pl.pallas_call(
        functools.partial(_dense_body, filt=filt, lags=lags),
        grid=grid,
        in_specs=[
            pl.BlockSpec((row_block, lags), row),
            pl.BlockSpec((row_block, filt), row),
            pl.BlockSpec((row_block, filt), row),
            pl.BlockSpec((row_block, GCOLS), row),
            pl.BlockSpec((row_block, GCOLS), row),
            pl.BlockSpec((row_block, GCOLS), row),
            pl.BlockSpec((row_block, GCOLS), row),
            pl.BlockSpec((lags, g4), rep),
            pl.BlockSpec((filt, g4), rep),
            pl.BlockSpec((GCOLS, g4), rep),
            pl.BlockSpec((GCOLS, g4), rep),
            pl.BlockSpec((GCOLS, g4), rep),
            pl.BlockSpec((1, g4), rep),
            pl.BlockSpec((filt, 1), rep),
            pl.BlockSpec((1, 1), rep),
        ],
        out_specs=[
            pl.BlockSpec((row_block, 1), row),
            pl.BlockSpec((row_block, filt), row),
            pl.BlockSpec((row_block, filt), row),
        ],
        out_shape=[
            jax.ShapeDtypeStruct((n_nodes, 1), jnp.float32),
            jax.ShapeDtypeStruct((n_nodes, filt), jnp.float32),
            jax.ShapeDtypeStruct((n_nodes, filt), jnp.float32),
        ],
    )


def kernel(x, edge_index, edge_weight, h_0, c_0, params):
    n, lags = x.shape
    filt = h_0.shape[1]
    e = edge_index.shape[1]

    t0 = h_0[:, :GCOLS]
    t1 = h_0[:, GCOLS:2 * GCOLS]
    t2 = jnp.concatenate(
        [x, jnp.ones((n, 1), jnp.float32),
         jnp.zeros((n, GCOLS - lags - 1), jnp.float32)], axis=1)

    rows = -(-e // CHUNK)
    blk = 2 * NS * ROWS_PER_STEP
    rows_p = -(-rows // blk) * blk
    e_pad = rows_p * CHUNK - e
    acc_rows = ((n + 1 + 8 * NS - 1) // (8 * NS)) * (8 * NS)
    pad_src = (jnp.arange(e_pad, dtype=jnp.int32) * 37) % n
    pad_dst = n + (jnp.arange(e_pad, dtype=jnp.int32) % (acc_rows - n))
    src2 = jnp.concatenate(
        [edge_index[0].astype(jnp.int32), pad_src]).reshape(rows_p, CHUNK)
    dst2 = jnp.concatenate(
        [edge_index[1].astype(jnp.int32), pad_dst]).reshape(rows_p, CHUNK)

    zeros_hbm = jnp.zeros((acc_rows, GCOLS), jnp.float32)

    a0, a1, a2a, a2b = _make_agg_kernel(n, rows_p, acc_rows)(
        t0, t1, t2, src2, dst2, zeros_hbm)

    gates_x = ['x_i', 'x_f', 'x_c', 'x_o']
    gates_h = ['h_i', 'h_f', 'h_c', 'h_o']

    def rel_w(p):
        return jnp.einsum('rb,bio->rio', p['comp'], p['basis'])[0]

    g4 = 4 * filt
    wx = jnp.concatenate([params[g]['root'] for g in gates_x], axis=1)
    wh = jnp.concatenate([params[g]['root'] for g in gates_h], axis=1)
    wax = jnp.concatenate([rel_w(params[g]) for g in gates_x], axis=1)
    wah = jnp.concatenate([rel_w(params[g]) for g in gates_h], axis=1)
    bias = jnp.concatenate(
        [params[gx]['bias'] + params[gh]['bias']
         for gx, gh in zip(gates_x, gates_h)])[None, :]
    w0 = wah[:GCOLS]
    w1 = wah[GCOLS:2 * GCOLS]
    w2 = jnp.concatenate(
        [wax, jnp.zeros((GCOLS - lags, g4), jnp.float32)], axis=0)

    row_block = 1000
    out, h_new, c = _make_dense_kernel(n, lags, filt, row_block)(
        x, h_0, c_0, a0, a1, a2a, a2b,
        wx, wh, w0, w1, w2, bias, params['lin_w'],
        params['lin_b'].reshape(1, 1))
    return (out, h_new, c)


# speedup vs baseline: 9.3521x; 1.7747x over previous
import functools

import jax
import jax.numpy as jnp
from jax import lax
from jax.experimental import pallas as pl
from jax.experimental.pallas import tpu as pltpu
from jax.experimental.pallas import tpu_sc as plsc

NC = 2
NS = 16
CHUNK = 128
ROWS_PER_STEP = 8
GCOLS = 16


def _make_agg_kernel(n_nodes, idx_rows, acc_rows):
    rows_per_sub = idx_rows // NS
    steps = rows_per_sub // ROWS_PER_STEP
    half_rows_per_sub = idx_rows // 2 // NS
    half_steps = half_rows_per_sub // ROWS_PER_STEP
    zrows = acc_rows // NS
    mesh = plsc.VectorSubcoreMesh(core_axis_name="c", subcore_axis_name="s")
    oshape = jax.ShapeDtypeStruct((acc_rows, GCOLS), jnp.float32)

    @functools.partial(
        pl.kernel,
        out_type=(oshape, oshape, oshape, oshape),
        mesh=mesh,
        scratch_types=[
            pltpu.VMEM((ROWS_PER_STEP, CHUNK), jnp.int32),
            pltpu.VMEM((ROWS_PER_STEP, CHUNK), jnp.int32),
            pltpu.VMEM((ROWS_PER_STEP, CHUNK, GCOLS), jnp.float32),
            pltpu.VMEM_SHARED((acc_rows, GCOLS), jnp.float32),
            pltpu.SemaphoreType.DMA,
            pltpu.SemaphoreType.DMA,
        ],
        compiler_params=pltpu.CompilerParams(use_tc_tiling_on_sc=False),
    )
    def agg_kernel(t0_hbm, t1_hbm, t2_hbm, src_hbm, dst_hbm, zeros_hbm,
                   out0, out1, out2a, out2b,
                   idx2s, idx2d, rb, acc, gsem, ssem):
        cid = lax.axis_index("c")
        sid = lax.axis_index("s")
        depth = 3

        def zero_acc():
            pltpu.sync_copy(zeros_hbm.at[pl.ds(sid * zrows, zrows)],
                            acc.at[pl.ds(sid * zrows, zrows)])

        def scan(table, row_base, nsteps):
            def step(b, carry):
                r0 = row_base + b * ROWS_PER_STEP
                pltpu.sync_copy(src_hbm.at[pl.ds(r0, ROWS_PER_STEP)], idx2s)
                pltpu.sync_copy(dst_hbm.at[pl.ds(r0, ROWS_PER_STEP)], idx2d)
                g = [pltpu.async_copy(table.at[idx2s.at[j]], rb.at[j], gsem)
                     for j in range(depth)]
                s = []
                for j in range(ROWS_PER_STEP):
                    g[j].wait()
                    if j + depth < ROWS_PER_STEP:
                        g.append(pltpu.async_copy(
                            table.at[idx2s.at[j + depth]], rb.at[j + depth],
                            gsem))
                    s.append(pltpu.async_copy(
                        rb.at[j], acc.at[idx2d.at[j]], ssem, add=True))
                for d in s:
                    d.wait()
                return carry
            lax.fori_loop(0, nsteps, step, 0)

        def dump(out):
            pltpu.sync_copy(acc.at[pl.ds(sid * zrows, zrows)],
                            out.at[pl.ds(sid * zrows, zrows)])

        zero_acc()
        plsc.subcore_barrier()

        @pl.when(cid == 0)
        def _():
            scan(t0_hbm, sid * rows_per_sub, steps)

        @pl.when(cid == 1)
        def _():
            scan(t1_hbm, sid * rows_per_sub, steps)
        plsc.subcore_barrier()

        @pl.when(cid == 0)
        def _():
            dump(out0)

        @pl.when(cid == 1)
        def _():
            dump(out1)
        plsc.subcore_barrier()

        zero_acc()
        plsc.subcore_barrier()
        scan(t2_hbm, cid * (idx_rows // 2) + sid * half_rows_per_sub,
             half_steps)
        plsc.subcore_barrier()

        @pl.when(cid == 0)
        def _():
            dump(out2a)

        @pl.when(cid == 1)
        def _():
            dump(out2b)

    return agg_kernel


def _dense_body(x_ref, h_ref, c0_ref, a0_ref, a1_ref, a2a_ref, a2b_ref,
                wx_ref, wh_ref, w0_ref, w1_ref, w2_ref, b_ref, lw_ref, lb_ref,
                out_ref, hn_ref, c_ref, *, filt, lags):
    def dot(a, b):
        return lax.dot_general(a, b, (((1,), (0,)), ((), ())),
                               precision=lax.Precision.HIGHEST,
                               preferred_element_type=jnp.float32)
    a0 = a0_ref[...]
    a1 = a1_ref[...]
    a2 = a2a_ref[...] + a2b_ref[...]
    inv = 1.0 / jnp.maximum(a2[:, lags:lags + 1], 1.0)
    pre = (dot(x_ref[...], wx_ref[...]) + dot(h_ref[...], wh_ref[...])
           + dot(a0 * inv, w0_ref[...]) + dot(a1 * inv, w1_ref[...])
           + dot(a2 * inv, w2_ref[...]) + b_ref[...])
    i_g = jax.nn.sigmoid(pre[:, 0 * filt:1 * filt])
    f_g = jax.nn.sigmoid(pre[:, 1 * filt:2 * filt])
    t_g = jnp.tanh(pre[:, 2 * filt:3 * filt])
    o_g = jax.nn.sigmoid(pre[:, 3 * filt:4 * filt])
    c = f_g * c0_ref[...] + i_g * t_g
    hn = o_g * jnp.tanh(c)
    h = jnp.maximum(hn, 0.0)
    out_ref[...] = dot(h, lw_ref[...]) + lb_ref[...]
    hn_ref[...] = hn
    c_ref[...] = c


def _make_dense_kernel(n_nodes, lags, filt, row_block):
    grid = (n_nodes // row_block,)
    g4 = 4 * filt
    row = lambda i: (i, 0)
    rep = lambda i: (0, 0)
    return pl.pallas_call(
        functools.partial(_dense_body, filt=filt, lags=lags),
        grid=grid,
        in_specs=[
            pl.BlockSpec((row_block, lags), row),
            pl.BlockSpec((row_block, filt), row),
            pl.BlockSpec((row_block, filt), row),
            pl.BlockSpec((row_block, GCOLS), row),
            pl.BlockSpec((row_block, GCOLS), row),
            pl.BlockSpec((row_block, GCOLS), row),
            pl.BlockSpec((row_block, GCOLS), row),
            pl.BlockSpec((lags, g4), rep),
            pl.BlockSpec((filt, g4), rep),
            pl.BlockSpec((GCOLS, g4), rep),
            pl.BlockSpec((GCOLS, g4), rep),
            pl.BlockSpec((GCOLS, g4), rep),
            pl.BlockSpec((1, g4), rep),
            pl.BlockSpec((filt, 1), rep),
            pl.BlockSpec((1, 1), rep),
        ],
        out_specs=[
            pl.BlockSpec((row_block, 1), row),
            pl.BlockSpec((row_block, filt), row),
            pl.BlockSpec((row_block, filt), row),
        ],
        out_shape=[
            jax.ShapeDtypeStruct((n_nodes, 1), jnp.float32),
            jax.ShapeDtypeStruct((n_nodes, filt), jnp.float32),
            jax.ShapeDtypeStruct((n_nodes, filt), jnp.float32),
        ],
    )


def kernel(x, edge_index, edge_weight, h_0, c_0, params):
    n, lags = x.shape
    filt = h_0.shape[1]
    e = edge_index.shape[1]

    t0 = h_0[:, :GCOLS]
    t1 = h_0[:, GCOLS:2 * GCOLS]
    t2 = jnp.concatenate(
        [x, jnp.ones((n, 1), jnp.float32),
         jnp.zeros((n, GCOLS - lags - 1), jnp.float32)], axis=1)

    rows = -(-e // CHUNK)
    blk = 2 * NS * ROWS_PER_STEP
    rows_p = -(-rows // blk) * blk
    e_pad = rows_p * CHUNK - e
    acc_rows = ((n + 1 + 8 * NS - 1) // (8 * NS)) * (8 * NS)
    pad_src = (jnp.arange(e_pad, dtype=jnp.int32) * 37) % n
    pad_dst = n + (jnp.arange(e_pad, dtype=jnp.int32) % (acc_rows - n))
    src2 = jnp.concatenate(
        [edge_index[0].astype(jnp.int32), pad_src]).reshape(rows_p, CHUNK)
    dst2 = jnp.concatenate(
        [edge_index[1].astype(jnp.int32), pad_dst]).reshape(rows_p, CHUNK)

    zeros_hbm = jnp.zeros((acc_rows, GCOLS), jnp.float32)

    a0, a1, a2a, a2b = _make_agg_kernel(n, rows_p, acc_rows)(
        t0, t1, t2, src2, dst2, zeros_hbm)

    gates_x = ['x_i', 'x_f', 'x_c', 'x_o']
    gates_h = ['h_i', 'h_f', 'h_c', 'h_o']

    def rel_w(p):
        return jnp.einsum('rb,bio->rio', p['comp'], p['basis'])[0]

    g4 = 4 * filt
    wx = jnp.concatenate([params[g]['root'] for g in gates_x], axis=1)
    wh = jnp.concatenate([params[g]['root'] for g in gates_h], axis=1)
    wax = jnp.concatenate([rel_w(params[g]) for g in gates_x], axis=1)
    wah = jnp.concatenate([rel_w(params[g]) for g in gates_h], axis=1)
    bias = jnp.concatenate(
        [params[gx]['bias'] + params[gh]['bias']
         for gx, gh in zip(gates_x, gates_h)])[None, :]
    w0 = wah[:GCOLS]
    w1 = wah[GCOLS:2 * GCOLS]
    w2 = jnp.concatenate(
        [wax, jnp.zeros((GCOLS - lags, g4), jnp.float32)], axis=0)

    row_block = 1000
    out, h_new, c = _make_dense_kernel(n, lags, filt, row_block)(
        x, h_0, c_0, a0, a1, a2a, a2b,
        wx, wh, w0, w1, w2, bias, params['lin_w'],
        params['lin_b'].reshape(1, 1))
    return (out, h_new, c)


# speedup vs baseline: 10.2462x; 1.0956x over previous
import functools

import jax
import jax.numpy as jnp
from jax import lax
from jax.experimental import pallas as pl
from jax.experimental.pallas import tpu as pltpu
from jax.experimental.pallas import tpu_sc as plsc

NC = 2
NS = 16
CHUNK = 128
ROWS_PER_STEP = 8
GCOLS = 16


def _make_agg_kernel(n_nodes, idx_rows, acc_rows):
    rows_per_sub = idx_rows // NS
    steps = rows_per_sub // ROWS_PER_STEP
    half_rows_per_sub = idx_rows // 2 // NS
    half_steps = half_rows_per_sub // ROWS_PER_STEP
    zrows = acc_rows // NS
    mesh = plsc.VectorSubcoreMesh(core_axis_name="c", subcore_axis_name="s")
    oshape = jax.ShapeDtypeStruct((acc_rows, GCOLS), jnp.float32)

    @functools.partial(
        pl.kernel,
        out_type=(oshape, oshape, oshape, oshape),
        mesh=mesh,
        scratch_types=[
            pltpu.VMEM((ROWS_PER_STEP, CHUNK), jnp.int32),
            pltpu.VMEM((ROWS_PER_STEP, CHUNK), jnp.int32),
            pltpu.VMEM((ROWS_PER_STEP, CHUNK, GCOLS), jnp.float32),
            pltpu.VMEM_SHARED((acc_rows, GCOLS), jnp.float32),
            pltpu.SemaphoreType.DMA,
            pltpu.SemaphoreType.DMA,
        ],
        compiler_params=pltpu.CompilerParams(use_tc_tiling_on_sc=False),
    )
    def agg_kernel(t0_hbm, t1_hbm, t2_hbm, src_hbm, dst_hbm, zeros_hbm,
                   out0, out1, out2a, out2b,
                   idx2s, idx2d, rb, acc, gsem, ssem):
        cid = lax.axis_index("c")
        sid = lax.axis_index("s")
        depth = 3

        def zero_acc():
            pltpu.sync_copy(zeros_hbm.at[pl.ds(sid * zrows, zrows)],
                            acc.at[pl.ds(sid * zrows, zrows)])

        def scan(table, row_base, nsteps):
            def step(b, carry):
                r0 = row_base + b * ROWS_PER_STEP
                pltpu.sync_copy(src_hbm.at[pl.ds(r0, ROWS_PER_STEP)], idx2s)
                pltpu.sync_copy(dst_hbm.at[pl.ds(r0, ROWS_PER_STEP)], idx2d)
                g = [pltpu.async_copy(table.at[idx2s.at[j]], rb.at[j], gsem)
                     for j in range(depth)]
                s = []
                for j in range(ROWS_PER_STEP):
                    g[j].wait()
                    if j + depth < ROWS_PER_STEP:
                        g.append(pltpu.async_copy(
                            table.at[idx2s.at[j + depth]], rb.at[j + depth],
                            gsem))
                    s.append(pltpu.async_copy(
                        rb.at[j], acc.at[idx2d.at[j]], ssem, add=True))
                for d in s:
                    d.wait()
                return carry
            lax.fori_loop(0, nsteps, step, 0)

        def dump(out):
            pltpu.sync_copy(acc.at[pl.ds(sid * zrows, zrows)],
                            out.at[pl.ds(sid * zrows, zrows)])

        zero_acc()
        plsc.subcore_barrier()

        @pl.when(cid == 0)
        def _():
            scan(t0_hbm, sid * rows_per_sub, steps)

        @pl.when(cid == 1)
        def _():
            scan(t1_hbm, sid * rows_per_sub, steps)
        plsc.subcore_barrier()

        @pl.when(cid == 0)
        def _():
            dump(out0)

        @pl.when(cid == 1)
        def _():
            dump(out1)
        plsc.subcore_barrier()

        zero_acc()
        plsc.subcore_barrier()
        scan(t2_hbm, cid * (idx_rows // 2) + sid * half_rows_per_sub,
             half_steps)
        plsc.subcore_barrier()

        @pl.when(cid == 0)
        def _():
            dump(out2a)

        @pl.when(cid == 1)
        def _():
            dump(out2b)

    return agg_kernel


def _dense_body(z_ref, c0_ref, w_ref, b_ref, lw_ref, lb_ref,
                out_ref, hn_ref, c_ref, *, filt, lags, kdim):
    def dot(a, b):
        return lax.dot_general(a, b, (((1,), (0,)), ((), ())),
                               precision=lax.Precision.HIGHEST,
                               preferred_element_type=jnp.float32)
    g4 = 4 * filt
    zc = z_ref[...]
    o = dot(zc, w_ref[...])
    c1 = lags + filt + 2 * GCOLS + lags
    c2 = c1 + GCOLS
    cnt = zc[:, c1:c1 + 1] + zc[:, c2:c2 + 1]
    inv = 1.0 / jnp.maximum(cnt, 1.0)
    pre = o[:, :g4] + inv * o[:, g4:2 * g4] + b_ref[...]
    i_g = jax.nn.sigmoid(pre[:, 0 * filt:1 * filt])
    f_g = jax.nn.sigmoid(pre[:, 1 * filt:2 * filt])
    t_g = jnp.tanh(pre[:, 2 * filt:3 * filt])
    o_g = jax.nn.sigmoid(pre[:, 3 * filt:4 * filt])
    c = f_g * c0_ref[...] + i_g * t_g
    hn = o_g * jnp.tanh(c)
    h = jnp.maximum(hn, 0.0)
    out_ref[...] = dot(h, lw_ref[...]) + lb_ref[...]
    hn_ref[...] = hn
    c_ref[...] = c


def _make_dense_kernel(n_nodes, lags, filt, kdim, row_block):
    grid = (n_nodes // row_block,)
    g4 = 4 * filt
    row = lambda i: (i, 0)
    rep = lambda i: (0, 0)
    return pl.pallas_call(
        functools.partial(_dense_body, filt=filt, lags=lags, kdim=kdim),
        grid=grid,
        in_specs=[
            pl.BlockSpec((row_block, kdim), row),
            pl.BlockSpec((row_block, filt), row),
            pl.BlockSpec((kdim, 2 * g4), rep),
            pl.BlockSpec((1, g4), rep),
            pl.BlockSpec((filt, 1), rep),
            pl.BlockSpec((1, 1), rep),
        ],
        out_specs=[
            pl.BlockSpec((row_block, 1), row),
            pl.BlockSpec((row_block, filt), row),
            pl.BlockSpec((row_block, filt), row),
        ],
        out_shape=[
            jax.ShapeDtypeStruct((n_nodes, 1), jnp.float32),
            jax.ShapeDtypeStruct((n_nodes, filt), jnp.float32),
            jax.ShapeDtypeStruct((n_nodes, filt), jnp.float32),
        ],
    )


def kernel(x, edge_index, edge_weight, h_0, c_0, params):
    n, lags = x.shape
    filt = h_0.shape[1]
    e = edge_index.shape[1]

    t0 = h_0[:, :GCOLS]
    t1 = h_0[:, GCOLS:2 * GCOLS]
    t2 = jnp.concatenate(
        [x, jnp.ones((n, 1), jnp.float32),
         jnp.zeros((n, GCOLS - lags - 1), jnp.float32)], axis=1)

    rows = -(-e // CHUNK)
    blk = 2 * NS * ROWS_PER_STEP
    rows_p = -(-rows // blk) * blk
    e_pad = rows_p * CHUNK - e
    acc_rows = ((n + 1 + 8 * NS - 1) // (8 * NS)) * (8 * NS)
    pad_src = (jnp.arange(e_pad, dtype=jnp.int32) * 37) % n
    pad_dst = n + (jnp.arange(e_pad, dtype=jnp.int32) % (acc_rows - n))
    src2 = jnp.concatenate(
        [edge_index[0].astype(jnp.int32), pad_src]).reshape(rows_p, CHUNK)
    dst2 = jnp.concatenate(
        [edge_index[1].astype(jnp.int32), pad_dst]).reshape(rows_p, CHUNK)

    zeros_hbm = jnp.zeros((acc_rows, GCOLS), jnp.float32)

    a0, a1, a2a, a2b = _make_agg_kernel(n, rows_p, acc_rows)(
        t0, t1, t2, src2, dst2, zeros_hbm)

    gates_x = ['x_i', 'x_f', 'x_c', 'x_o']
    gates_h = ['h_i', 'h_f', 'h_c', 'h_o']

    def rel_w(p):
        return jnp.einsum('rb,bio->rio', p['comp'], p['basis'])[0]

    g4 = 4 * filt
    wx = jnp.concatenate([params[g]['root'] for g in gates_x], axis=1)
    wh = jnp.concatenate([params[g]['root'] for g in gates_h], axis=1)
    wax = jnp.concatenate([rel_w(params[g]) for g in gates_x], axis=1)
    wah = jnp.concatenate([rel_w(params[g]) for g in gates_h], axis=1)
    bias = jnp.concatenate(
        [params[gx]['bias'] + params[gh]['bias']
         for gx, gh in zip(gates_x, gates_h)])[None, :]
    w2 = jnp.concatenate(
        [wax, jnp.zeros((GCOLS - lags, g4), jnp.float32)], axis=0)

    kdim = lags + filt + 4 * GCOLS
    z = jnp.concatenate(
        [x, h_0, a0[:n], a1[:n], a2a[:n], a2b[:n]], axis=1)
    w_left = jnp.concatenate([wx, wh, jnp.zeros((4 * GCOLS, g4))], axis=0)
    w_right = jnp.concatenate(
        [jnp.zeros((lags + filt, g4)), wah[:GCOLS], wah[GCOLS:2 * GCOLS],
         w2, w2], axis=0)
    w256 = jnp.concatenate([w_left, w_right], axis=1)

    row_block = 2000
    out, h_new, c = _make_dense_kernel(n, lags, filt, kdim, row_block)(
        z, c_0, w256, bias, params['lin_w'], params['lin_b'].reshape(1, 1))
    return (out, h_new, c)


# speedup vs baseline: 11.9686x; 1.1681x over previous
import functools

import jax
import jax.numpy as jnp
from jax import lax
from jax.experimental import pallas as pl
from jax.experimental.pallas import tpu as pltpu
from jax.experimental.pallas import tpu_sc as plsc

NC = 2
NS = 16
CHUNK = 128
ROWS_PER_STEP = 8
GCOLS = 16


def _make_agg_kernel(n_nodes, idx_rows, acc_rows):
    rows_per_sub = idx_rows // NS
    steps = rows_per_sub // ROWS_PER_STEP
    half_rows_per_sub = idx_rows // 2 // NS
    half_steps = half_rows_per_sub // ROWS_PER_STEP
    zrows = acc_rows // NS
    mesh = plsc.VectorSubcoreMesh(core_axis_name="c", subcore_axis_name="s")
    oshape = jax.ShapeDtypeStruct((acc_rows, GCOLS), jnp.float32)

    @functools.partial(
        pl.kernel,
        out_type=(oshape, oshape, oshape, oshape),
        mesh=mesh,
        scratch_types=[
            pltpu.VMEM((ROWS_PER_STEP, CHUNK), jnp.int32),
            pltpu.VMEM((ROWS_PER_STEP, CHUNK), jnp.int32),
            pltpu.VMEM((ROWS_PER_STEP, CHUNK, GCOLS), jnp.float32),
            pltpu.VMEM_SHARED((acc_rows, GCOLS), jnp.float32),
            pltpu.SemaphoreType.DMA,
            pltpu.SemaphoreType.DMA,
        ],
        compiler_params=pltpu.CompilerParams(use_tc_tiling_on_sc=False),
    )
    def agg_kernel(t0_hbm, t1_hbm, t2_hbm, src_hbm, dst_hbm, zeros_hbm,
                   out0, out1, out2a, out2b,
                   idx2s, idx2d, rb, acc, gsem, ssem):
        cid = lax.axis_index("c")
        sid = lax.axis_index("s")
        depth = 3

        def zero_acc():
            pltpu.sync_copy(zeros_hbm.at[pl.ds(sid * zrows, zrows)],
                            acc.at[pl.ds(sid * zrows, zrows)])

        def scan(table, row_base, nsteps):
            def step(b, carry):
                r0 = row_base + b * ROWS_PER_STEP
                pltpu.sync_copy(src_hbm.at[pl.ds(r0, ROWS_PER_STEP)], idx2s)
                pltpu.sync_copy(dst_hbm.at[pl.ds(r0, ROWS_PER_STEP)], idx2d)
                g = [pltpu.async_copy(table.at[idx2s.at[j]], rb.at[j], gsem)
                     for j in range(depth)]
                s = []
                for j in range(ROWS_PER_STEP):
                    g[j].wait()
                    if j + depth < ROWS_PER_STEP:
                        g.append(pltpu.async_copy(
                            table.at[idx2s.at[j + depth]], rb.at[j + depth],
                            gsem))
                    s.append(pltpu.async_copy(
                        rb.at[j], acc.at[idx2d.at[j]], ssem, add=True))
                for d in s:
                    d.wait()
                return carry
            lax.fori_loop(0, nsteps, step, 0)

        def dump(out):
            pltpu.sync_copy(acc.at[pl.ds(sid * zrows, zrows)],
                            out.at[pl.ds(sid * zrows, zrows)])

        zero_acc()
        plsc.subcore_barrier()

        @pl.when(cid == 0)
        def _():
            scan(t0_hbm, sid * rows_per_sub, steps)

        @pl.when(cid == 1)
        def _():
            scan(t1_hbm, sid * rows_per_sub, steps)
        plsc.subcore_barrier()

        @pl.when(cid == 0)
        def _():
            dump(out0)

        @pl.when(cid == 1)
        def _():
            dump(out1)
        plsc.subcore_barrier()

        zero_acc()
        plsc.subcore_barrier()
        scan(t2_hbm, cid * (idx_rows // 2) + sid * half_rows_per_sub,
             half_steps)
        plsc.subcore_barrier()

        @pl.when(cid == 0)
        def _():
            dump(out2a)

        @pl.when(cid == 1)
        def _():
            dump(out2b)

    return agg_kernel


def _dense_body(x_ref, h_ref, c0_ref, a0_ref, a1_ref, a2a_ref, a2b_ref,
                w_ref, b_ref, lw_ref, lb_ref,
                out_ref, hn_ref, c_ref, *, filt, lags, kdim):
    def dot(a, b):
        return lax.dot_general(a, b, (((1,), (0,)), ((), ())),
                               precision=lax.Precision.HIGHEST,
                               preferred_element_type=jnp.float32)
    g4 = 4 * filt
    a2a = a2a_ref[...]
    a2b = a2b_ref[...]
    zc = jnp.concatenate(
        [x_ref[...], h_ref[...], a0_ref[...], a1_ref[...], a2a, a2b], axis=1)
    o = dot(zc, w_ref[...])
    cnt = a2a[:, lags:lags + 1] + a2b[:, lags:lags + 1]
    inv = 1.0 / jnp.maximum(cnt, 1.0)
    pre = o[:, :g4] + inv * o[:, g4:2 * g4] + b_ref[...]
    i_g = jax.nn.sigmoid(pre[:, 0 * filt:1 * filt])
    f_g = jax.nn.sigmoid(pre[:, 1 * filt:2 * filt])
    t_g = jnp.tanh(pre[:, 2 * filt:3 * filt])
    o_g = jax.nn.sigmoid(pre[:, 3 * filt:4 * filt])
    c = f_g * c0_ref[...] + i_g * t_g
    hn = o_g * jnp.tanh(c)
    h = jnp.maximum(hn, 0.0)
    out_ref[...] = dot(h, lw_ref[...]) + lb_ref[...]
    hn_ref[...] = hn
    c_ref[...] = c


def _make_dense_kernel(n_nodes, lags, filt, kdim, row_block):
    grid = (n_nodes // row_block,)
    g4 = 4 * filt
    row = lambda i: (i, 0)
    rep = lambda i: (0, 0)
    return pl.pallas_call(
        functools.partial(_dense_body, filt=filt, lags=lags, kdim=kdim),
        grid=grid,
        in_specs=[
            pl.BlockSpec((row_block, lags), row),
            pl.BlockSpec((row_block, filt), row),
            pl.BlockSpec((row_block, filt), row),
            pl.BlockSpec((row_block, GCOLS), row),
            pl.BlockSpec((row_block, GCOLS), row),
            pl.BlockSpec((row_block, GCOLS), row),
            pl.BlockSpec((row_block, GCOLS), row),
            pl.BlockSpec((kdim, 2 * g4), rep),
            pl.BlockSpec((1, g4), rep),
            pl.BlockSpec((filt, 1), rep),
            pl.BlockSpec((1, 1), rep),
        ],
        out_specs=[
            pl.BlockSpec((row_block, 1), row),
            pl.BlockSpec((row_block, filt), row),
            pl.BlockSpec((row_block, filt), row),
        ],
        out_shape=[
            jax.ShapeDtypeStruct((n_nodes, 1), jnp.float32),
            jax.ShapeDtypeStruct((n_nodes, filt), jnp.float32),
            jax.ShapeDtypeStruct((n_nodes, filt), jnp.float32),
        ],
    )


def kernel(x, edge_index, edge_weight, h_0, c_0, params):
    n, lags = x.shape
    filt = h_0.shape[1]
    e = edge_index.shape[1]

    t0 = h_0[:, :GCOLS]
    t1 = h_0[:, GCOLS:2 * GCOLS]
    t2 = jnp.concatenate(
        [x, jnp.ones((n, 1), jnp.float32),
         jnp.zeros((n, GCOLS - lags - 1), jnp.float32)], axis=1)

    rows = -(-e // CHUNK)
    blk = 2 * NS * ROWS_PER_STEP
    rows_p = -(-rows // blk) * blk
    e_pad = rows_p * CHUNK - e
    acc_rows = ((n + 1 + 8 * NS - 1) // (8 * NS)) * (8 * NS)
    pad_src = (jnp.arange(e_pad, dtype=jnp.int32) * 37) % n
    pad_dst = n + (jnp.arange(e_pad, dtype=jnp.int32) % (acc_rows - n))
    src2 = jnp.concatenate(
        [edge_index[0].astype(jnp.int32), pad_src]).reshape(rows_p, CHUNK)
    dst2 = jnp.concatenate(
        [edge_index[1].astype(jnp.int32), pad_dst]).reshape(rows_p, CHUNK)

    zeros_hbm = jnp.zeros((acc_rows, GCOLS), jnp.float32)

    a0, a1, a2a, a2b = _make_agg_kernel(n, rows_p, acc_rows)(
        t0, t1, t2, src2, dst2, zeros_hbm)

    gates_x = ['x_i', 'x_f', 'x_c', 'x_o']
    gates_h = ['h_i', 'h_f', 'h_c', 'h_o']

    def rel_w(p):
        return jnp.einsum('rb,bio->rio', p['comp'], p['basis'])[0]

    g4 = 4 * filt
    wx = jnp.concatenate([params[g]['root'] for g in gates_x], axis=1)
    wh = jnp.concatenate([params[g]['root'] for g in gates_h], axis=1)
    wax = jnp.concatenate([rel_w(params[g]) for g in gates_x], axis=1)
    wah = jnp.concatenate([rel_w(params[g]) for g in gates_h], axis=1)
    bias = jnp.concatenate(
        [params[gx]['bias'] + params[gh]['bias']
         for gx, gh in zip(gates_x, gates_h)])[None, :]
    w2 = jnp.concatenate(
        [wax, jnp.zeros((GCOLS - lags, g4), jnp.float32)], axis=0)

    kdim = lags + filt + 4 * GCOLS
    w_left = jnp.concatenate([wx, wh, jnp.zeros((4 * GCOLS, g4))], axis=0)
    w_right = jnp.concatenate(
        [jnp.zeros((lags + filt, g4)), wah[:GCOLS], wah[GCOLS:2 * GCOLS],
         w2, w2], axis=0)
    w256 = jnp.concatenate([w_left, w_right], axis=1)

    row_block = 2000
    out, h_new, c = _make_dense_kernel(n, lags, filt, kdim, row_block)(
        x, h_0, c_0, a0, a1, a2a, a2b,
        w256, bias, params['lin_w'], params['lin_b'].reshape(1, 1))
    return (out, h_new, c)


# speedup vs baseline: 13.3464x; 1.1151x over previous
import functools

import jax
import jax.numpy as jnp
from jax import lax
from jax.experimental import pallas as pl
from jax.experimental.pallas import tpu as pltpu
from jax.experimental.pallas import tpu_sc as plsc

NC = 2
NS = 16
CHUNK = 128
ROWS_PER_STEP = 8
GCOLS = 16


def _make_agg_kernel(n_nodes, idx_rows, acc_rows):
    rows_per_sub = idx_rows // NS
    steps = rows_per_sub // ROWS_PER_STEP
    half_rows_per_sub = idx_rows // 2 // NS
    half_steps = half_rows_per_sub // ROWS_PER_STEP
    zrows = acc_rows // NS
    mesh = plsc.VectorSubcoreMesh(core_axis_name="c", subcore_axis_name="s")
    oshape = jax.ShapeDtypeStruct((acc_rows, GCOLS), jnp.float32)

    @functools.partial(
        pl.kernel,
        out_type=(oshape, oshape, oshape, oshape),
        mesh=mesh,
        scratch_types=[
            pltpu.VMEM((2 * ROWS_PER_STEP, CHUNK), jnp.int32),
            pltpu.VMEM((2 * ROWS_PER_STEP, CHUNK), jnp.int32),
            pltpu.VMEM((ROWS_PER_STEP, CHUNK, GCOLS), jnp.float32),
            pltpu.VMEM_SHARED((acc_rows, GCOLS), jnp.float32),
            pltpu.SemaphoreType.DMA,
            pltpu.SemaphoreType.DMA,
            pltpu.SemaphoreType.DMA,
        ],
        compiler_params=pltpu.CompilerParams(use_tc_tiling_on_sc=False),
    )
    def agg_kernel(t0_hbm, t1_hbm, t2_hbm, idxb_hbm, zeros_hbm,
                   out0, out1, out2a, out2b,
                   ib0, ib1, rb, acc, gsem, ssem, isem):
        cid = lax.axis_index("c")
        sid = lax.axis_index("s")
        depth = 3
        blk = 2 * ROWS_PER_STEP

        def zero_acc():
            pltpu.sync_copy(zeros_hbm.at[pl.ds(sid * zrows, zrows)],
                            acc.at[pl.ds(sid * zrows, zrows)])

        def scan(table, step_base, nsteps):
            pltpu.async_copy(idxb_hbm.at[pl.ds(step_base * blk, blk)],
                             ib0, isem)

            def run_step(b, cur, nxt, prefetch):
                pltpu.make_async_copy(
                    idxb_hbm.at[pl.ds(0, blk)], cur, isem).wait()
                if prefetch:
                    @pl.when(b + 1 < nsteps)
                    def _():
                        pltpu.async_copy(
                            idxb_hbm.at[pl.ds((step_base + b + 1) * blk, blk)],
                            nxt, isem)
                g = [pltpu.async_copy(table.at[cur.at[j]], rb.at[j], gsem)
                     for j in range(depth)]
                s = []
                for j in range(ROWS_PER_STEP):
                    g[j].wait()
                    if j + depth < ROWS_PER_STEP:
                        g.append(pltpu.async_copy(
                            table.at[cur.at[j + depth]], rb.at[j + depth],
                            gsem))
                    s.append(pltpu.async_copy(
                        rb.at[j], acc.at[cur.at[ROWS_PER_STEP + j]], ssem,
                        add=True))
                for d in s:
                    d.wait()

            def double_step(b2, carry):
                run_step(2 * b2, ib0, ib1, True)
                run_step(2 * b2 + 1, ib1, ib0, True)
                return carry
            lax.fori_loop(0, nsteps // 2, double_step, 0)
            if nsteps % 2:
                run_step(nsteps - 1, ib0, ib1, False)

        def dump(out):
            pltpu.sync_copy(acc.at[pl.ds(sid * zrows, zrows)],
                            out.at[pl.ds(sid * zrows, zrows)])

        zero_acc()
        plsc.subcore_barrier()

        @pl.when(cid == 0)
        def _():
            scan(t0_hbm, sid * steps, steps)

        @pl.when(cid == 1)
        def _():
            scan(t1_hbm, sid * steps, steps)
        plsc.subcore_barrier()

        @pl.when(cid == 0)
        def _():
            dump(out0)

        @pl.when(cid == 1)
        def _():
            dump(out1)
        plsc.subcore_barrier()

        zero_acc()
        plsc.subcore_barrier()
        scan(t2_hbm,
             cid * (idx_rows // 2 // ROWS_PER_STEP) + sid * half_steps,
             half_steps)
        plsc.subcore_barrier()

        @pl.when(cid == 0)
        def _():
            dump(out2a)

        @pl.when(cid == 1)
        def _():
            dump(out2b)

    return agg_kernel


def _dense_body(x_ref, h_ref, c0_ref, a0_ref, a1_ref, a2a_ref, a2b_ref,
                w_ref, b_ref, lw_ref, lb_ref,
                out_ref, hn_ref, c_ref, *, filt, lags, kdim):
    def dot(a, b):
        return lax.dot_general(a, b, (((1,), (0,)), ((), ())),
                               precision=lax.Precision.HIGHEST,
                               preferred_element_type=jnp.float32)
    g4 = 4 * filt
    a2a = a2a_ref[...]
    a2b = a2b_ref[...]
    zc = jnp.concatenate(
        [x_ref[...], h_ref[...], a0_ref[...], a1_ref[...], a2a, a2b], axis=1)
    o = dot(zc, w_ref[...])
    cnt = a2a[:, lags:lags + 1] + a2b[:, lags:lags + 1]
    inv = 1.0 / jnp.maximum(cnt, 1.0)
    pre = o[:, :g4] + inv * o[:, g4:2 * g4] + b_ref[...]
    i_g = jax.nn.sigmoid(pre[:, 0 * filt:1 * filt])
    f_g = jax.nn.sigmoid(pre[:, 1 * filt:2 * filt])
    t_g = jnp.tanh(pre[:, 2 * filt:3 * filt])
    o_g = jax.nn.sigmoid(pre[:, 3 * filt:4 * filt])
    c = f_g * c0_ref[...] + i_g * t_g
    hn = o_g * jnp.tanh(c)
    h = jnp.maximum(hn, 0.0)
    out_ref[...] = dot(h, lw_ref[...]) + lb_ref[...]
    hn_ref[...] = hn
    c_ref[...] = c


def _make_dense_kernel(n_nodes, lags, filt, kdim, row_block):
    grid = (n_nodes // row_block,)
    g4 = 4 * filt
    row = lambda i: (i, 0)
    rep = lambda i: (0, 0)
    return pl.pallas_call(
        functools.partial(_dense_body, filt=filt, lags=lags, kdim=kdim),
        grid=grid,
        in_specs=[
            pl.BlockSpec((row_block, lags), row),
            pl.BlockSpec((row_block, filt), row),
            pl.BlockSpec((row_block, filt), row),
            pl.BlockSpec((row_block, GCOLS), row),
            pl.BlockSpec((row_block, GCOLS), row),
            pl.BlockSpec((row_block, GCOLS), row),
            pl.BlockSpec((row_block, GCOLS), row),
            pl.BlockSpec((kdim, 2 * g4), rep),
            pl.BlockSpec((1, g4), rep),
            pl.BlockSpec((filt, 1), rep),
            pl.BlockSpec((1, 1), rep),
        ],
        out_specs=[
            pl.BlockSpec((row_block, 1), row),
            pl.BlockSpec((row_block, filt), row),
            pl.BlockSpec((row_block, filt), row),
        ],
        out_shape=[
            jax.ShapeDtypeStruct((n_nodes, 1), jnp.float32),
            jax.ShapeDtypeStruct((n_nodes, filt), jnp.float32),
            jax.ShapeDtypeStruct((n_nodes, filt), jnp.float32),
        ],
    )


def kernel(x, edge_index, edge_weight, h_0, c_0, params):
    n, lags = x.shape
    filt = h_0.shape[1]
    e = edge_index.shape[1]

    t0 = h_0[:, :GCOLS]
    t1 = h_0[:, GCOLS:2 * GCOLS]
    t2 = jnp.concatenate(
        [x, jnp.ones((n, 1), jnp.float32),
         jnp.zeros((n, GCOLS - lags - 1), jnp.float32)], axis=1)

    rows = -(-e // CHUNK)
    blk = 2 * NS * ROWS_PER_STEP
    rows_p = -(-rows // blk) * blk
    e_pad = rows_p * CHUNK - e
    acc_rows = ((n + 1 + 8 * NS - 1) // (8 * NS)) * (8 * NS)
    pad_src = (jnp.arange(e_pad, dtype=jnp.int32) * 37) % n
    pad_dst = n + (jnp.arange(e_pad, dtype=jnp.int32) % (acc_rows - n))
    src2 = jnp.concatenate(
        [edge_index[0].astype(jnp.int32), pad_src]).reshape(
            rows_p // ROWS_PER_STEP, ROWS_PER_STEP, CHUNK)
    dst2 = jnp.concatenate(
        [edge_index[1].astype(jnp.int32), pad_dst]).reshape(
            rows_p // ROWS_PER_STEP, ROWS_PER_STEP, CHUNK)
    idxb = jnp.concatenate([src2, dst2], axis=1).reshape(2 * rows_p, CHUNK)

    zeros_hbm = jnp.zeros((acc_rows, GCOLS), jnp.float32)

    a0, a1, a2a, a2b = _make_agg_kernel(n, rows_p, acc_rows)(
        t0, t1, t2, idxb, zeros_hbm)

    gates_x = ['x_i', 'x_f', 'x_c', 'x_o']
    gates_h = ['h_i', 'h_f', 'h_c', 'h_o']

    def rel_w(p):
        return jnp.einsum('rb,bio->rio', p['comp'], p['basis'])[0]

    g4 = 4 * filt
    wx = jnp.concatenate([params[g]['root'] for g in gates_x], axis=1)
    wh = jnp.concatenate([params[g]['root'] for g in gates_h], axis=1)
    wax = jnp.concatenate([rel_w(params[g]) for g in gates_x], axis=1)
    wah = jnp.concatenate([rel_w(params[g]) for g in gates_h], axis=1)
    bias = jnp.concatenate(
        [params[gx]['bias'] + params[gh]['bias']
         for gx, gh in zip(gates_x, gates_h)])[None, :]
    w2 = jnp.concatenate(
        [wax, jnp.zeros((GCOLS - lags, g4), jnp.float32)], axis=0)

    kdim = lags + filt + 4 * GCOLS
    w_left = jnp.concatenate([wx, wh, jnp.zeros((4 * GCOLS, g4))], axis=0)
    w_right = jnp.concatenate(
        [jnp.zeros((lags + filt, g4)), wah[:GCOLS], wah[GCOLS:2 * GCOLS],
         w2, w2], axis=0)
    w256 = jnp.concatenate([w_left, w_right], axis=1)

    row_block = 2000
    out, h_new, c = _make_dense_kernel(n, lags, filt, kdim, row_block)(
        x, h_0, c_0, a0, a1, a2a, a2b,
        w256, bias, params['lin_w'], params['lin_b'].reshape(1, 1))
    return (out, h_new, c)


# speedup vs baseline: 14.2372x; 1.0667x over previous
import functools

import jax
import jax.numpy as jnp
from jax import lax
from jax.experimental import pallas as pl
from jax.experimental.pallas import tpu as pltpu
from jax.experimental.pallas import tpu_sc as plsc

NC = 2
NS = 16
CHUNK = 128
ROWS_PER_STEP = 8
GCOLS = 16


def _make_agg_kernel(n_nodes, idx_rows, acc_rows):
    rows_per_sub = idx_rows // NS
    steps = rows_per_sub // ROWS_PER_STEP
    half_rows_per_sub = idx_rows // 2 // NS
    half_steps = half_rows_per_sub // ROWS_PER_STEP
    zrows = acc_rows // NS
    mesh = plsc.VectorSubcoreMesh(core_axis_name="c", subcore_axis_name="s")
    oshape = jax.ShapeDtypeStruct((acc_rows, GCOLS), jnp.float32)

    @functools.partial(
        pl.kernel,
        out_type=(oshape, oshape, oshape, oshape),
        mesh=mesh,
        scratch_types=[
            pltpu.VMEM((2 * ROWS_PER_STEP, CHUNK), jnp.int32),
            pltpu.VMEM((2 * ROWS_PER_STEP, CHUNK), jnp.int32),
            pltpu.VMEM((ROWS_PER_STEP, CHUNK, GCOLS), jnp.float32),
            pltpu.VMEM_SHARED((acc_rows, GCOLS), jnp.float32),
            pltpu.SemaphoreType.DMA,
            pltpu.SemaphoreType.DMA,
            pltpu.SemaphoreType.DMA,
        ],
        compiler_params=pltpu.CompilerParams(use_tc_tiling_on_sc=False),
    )
    def agg_kernel(t0_hbm, t1_hbm, t2_hbm, idxb_hbm, zeros_hbm,
                   out0, out1, out2a, out2b,
                   ib0, ib1, rb, acc, gsem, ssem, isem):
        cid = lax.axis_index("c")
        sid = lax.axis_index("s")
        depth = 4
        blk = 2 * ROWS_PER_STEP

        def zero_acc():
            pltpu.sync_copy(zeros_hbm.at[pl.ds(sid * zrows, zrows)],
                            acc.at[pl.ds(sid * zrows, zrows)])

        def scan(table, step_base, nsteps):
            pltpu.async_copy(idxb_hbm.at[pl.ds(step_base * blk, blk)],
                             ib0, isem)

            def run_step(b, cur, nxt, prefetch):
                pltpu.make_async_copy(
                    idxb_hbm.at[pl.ds(0, blk)], cur, isem).wait()
                if prefetch:
                    @pl.when(b + 1 < nsteps)
                    def _():
                        pltpu.async_copy(
                            idxb_hbm.at[pl.ds((step_base + b + 1) * blk, blk)],
                            nxt, isem)
                g = [pltpu.async_copy(table.at[cur.at[j]], rb.at[j], gsem)
                     for j in range(depth)]
                s = []
                for j in range(ROWS_PER_STEP):
                    g[j].wait()
                    if j + depth < ROWS_PER_STEP:
                        g.append(pltpu.async_copy(
                            table.at[cur.at[j + depth]], rb.at[j + depth],
                            gsem))
                    s.append(pltpu.async_copy(
                        rb.at[j], acc.at[cur.at[ROWS_PER_STEP + j]], ssem,
                        add=True))
                for d in s:
                    d.wait()

            def double_step(b2, carry):
                run_step(2 * b2, ib0, ib1, True)
                run_step(2 * b2 + 1, ib1, ib0, True)
                return carry
            lax.fori_loop(0, nsteps // 2, double_step, 0)
            if nsteps % 2:
                run_step(nsteps - 1, ib0, ib1, False)

        def dump(out):
            pltpu.sync_copy(acc.at[pl.ds(sid * zrows, zrows)],
                            out.at[pl.ds(sid * zrows, zrows)])

        zero_acc()
        plsc.subcore_barrier()

        @pl.when(cid == 0)
        def _():
            scan(t0_hbm, sid * steps, steps)

        @pl.when(cid == 1)
        def _():
            scan(t1_hbm, sid * steps, steps)
        plsc.subcore_barrier()

        @pl.when(cid == 0)
        def _():
            dump(out0)

        @pl.when(cid == 1)
        def _():
            dump(out1)
        plsc.subcore_barrier()

        zero_acc()
        plsc.subcore_barrier()
        scan(t2_hbm,
             cid * (idx_rows // 2 // ROWS_PER_STEP) + sid * half_steps,
             half_steps)
        plsc.subcore_barrier()

        @pl.when(cid == 0)
        def _():
            dump(out2a)

        @pl.when(cid == 1)
        def _():
            dump(out2b)

    return agg_kernel


def _dense_body(x_ref, h_ref, c0_ref, a0_ref, a1_ref, a2a_ref, a2b_ref,
                w_ref, b_ref, lw_ref, lb_ref,
                out_ref, hn_ref, c_ref, *, filt, lags, kdim):
    def dot(a, b):
        return lax.dot_general(a, b, (((1,), (0,)), ((), ())),
                               precision=lax.Precision.HIGHEST,
                               preferred_element_type=jnp.float32)
    g4 = 4 * filt
    a2a = a2a_ref[...]
    a2b = a2b_ref[...]
    zc = jnp.concatenate(
        [x_ref[...], h_ref[...], a0_ref[...], a1_ref[...], a2a, a2b], axis=1)
    o = dot(zc, w_ref[...])
    cnt = a2a[:, lags:lags + 1] + a2b[:, lags:lags + 1]
    inv = 1.0 / jnp.maximum(cnt, 1.0)
    pre = o[:, :g4] + inv * o[:, g4:2 * g4] + b_ref[...]
    i_g = jax.nn.sigmoid(pre[:, 0 * filt:1 * filt])
    f_g = jax.nn.sigmoid(pre[:, 1 * filt:2 * filt])
    t_g = jnp.tanh(pre[:, 2 * filt:3 * filt])
    o_g = jax.nn.sigmoid(pre[:, 3 * filt:4 * filt])
    c = f_g * c0_ref[...] + i_g * t_g
    hn = o_g * jnp.tanh(c)
    h = jnp.maximum(hn, 0.0)
    out_ref[...] = dot(h, lw_ref[...]) + lb_ref[...]
    hn_ref[...] = hn
    c_ref[...] = c


def _make_dense_kernel(n_nodes, lags, filt, kdim, row_block):
    grid = (n_nodes // row_block,)
    g4 = 4 * filt
    row = lambda i: (i, 0)
    rep = lambda i: (0, 0)
    return pl.pallas_call(
        functools.partial(_dense_body, filt=filt, lags=lags, kdim=kdim),
        grid=grid,
        in_specs=[
            pl.BlockSpec((row_block, lags), row),
            pl.BlockSpec((row_block, filt), row),
            pl.BlockSpec((row_block, filt), row),
            pl.BlockSpec((row_block, GCOLS), row),
            pl.BlockSpec((row_block, GCOLS), row),
            pl.BlockSpec((row_block, GCOLS), row),
            pl.BlockSpec((row_block, GCOLS), row),
            pl.BlockSpec((kdim, 2 * g4), rep),
            pl.BlockSpec((1, g4), rep),
            pl.BlockSpec((filt, 1), rep),
            pl.BlockSpec((1, 1), rep),
        ],
        out_specs=[
            pl.BlockSpec((row_block, 1), row),
            pl.BlockSpec((row_block, filt), row),
            pl.BlockSpec((row_block, filt), row),
        ],
        out_shape=[
            jax.ShapeDtypeStruct((n_nodes, 1), jnp.float32),
            jax.ShapeDtypeStruct((n_nodes, filt), jnp.float32),
            jax.ShapeDtypeStruct((n_nodes, filt), jnp.float32),
        ],
    )


def kernel(x, edge_index, edge_weight, h_0, c_0, params):
    n, lags = x.shape
    filt = h_0.shape[1]
    e = edge_index.shape[1]

    t0 = h_0[:, :GCOLS]
    t1 = h_0[:, GCOLS:2 * GCOLS]
    t2 = jnp.concatenate(
        [x, jnp.ones((n, 1), jnp.float32),
         jnp.zeros((n, GCOLS - lags - 1), jnp.float32)], axis=1)

    rows = -(-e // CHUNK)
    blk = 2 * NS * ROWS_PER_STEP
    rows_p = -(-rows // blk) * blk
    e_pad = rows_p * CHUNK - e
    acc_rows = ((n + 1 + 8 * NS - 1) // (8 * NS)) * (8 * NS)
    pad_src = (jnp.arange(e_pad, dtype=jnp.int32) * 37) % n
    pad_dst = n + (jnp.arange(e_pad, dtype=jnp.int32) % (acc_rows - n))
    src2 = jnp.concatenate(
        [edge_index[0].astype(jnp.int32), pad_src]).reshape(
            rows_p // ROWS_PER_STEP, ROWS_PER_STEP, CHUNK)
    dst2 = jnp.concatenate(
        [edge_index[1].astype(jnp.int32), pad_dst]).reshape(
            rows_p // ROWS_PER_STEP, ROWS_PER_STEP, CHUNK)
    idxb = jnp.concatenate([src2, dst2], axis=1).reshape(2 * rows_p, CHUNK)

    zeros_hbm = jnp.zeros((acc_rows, GCOLS), jnp.float32)

    a0, a1, a2a, a2b = _make_agg_kernel(n, rows_p, acc_rows)(
        t0, t1, t2, idxb, zeros_hbm)

    gates_x = ['x_i', 'x_f', 'x_c', 'x_o']
    gates_h = ['h_i', 'h_f', 'h_c', 'h_o']

    def rel_w(p):
        return jnp.einsum('rb,bio->rio', p['comp'], p['basis'])[0]

    g4 = 4 * filt
    wx = jnp.concatenate([params[g]['root'] for g in gates_x], axis=1)
    wh = jnp.concatenate([params[g]['root'] for g in gates_h], axis=1)
    wax = jnp.concatenate([rel_w(params[g]) for g in gates_x], axis=1)
    wah = jnp.concatenate([rel_w(params[g]) for g in gates_h], axis=1)
    bias = jnp.concatenate(
        [params[gx]['bias'] + params[gh]['bias']
         for gx, gh in zip(gates_x, gates_h)])[None, :]
    w2 = jnp.concatenate(
        [wax, jnp.zeros((GCOLS - lags, g4), jnp.float32)], axis=0)

    kdim = lags + filt + 4 * GCOLS
    w_left = jnp.concatenate([wx, wh, jnp.zeros((4 * GCOLS, g4))], axis=0)
    w_right = jnp.concatenate(
        [jnp.zeros((lags + filt, g4)), wah[:GCOLS], wah[GCOLS:2 * GCOLS],
         w2, w2], axis=0)
    w256 = jnp.concatenate([w_left, w_right], axis=1)

    row_block = 2000
    out, h_new, c = _make_dense_kernel(n, lags, filt, kdim, row_block)(
        x, h_0, c_0, a0, a1, a2a, a2b,
        w256, bias, params['lin_w'], params['lin_b'].reshape(1, 1))
    return (out, h_new, c)


# speedup vs baseline: 15.1671x; 1.0653x over previous
import functools

import jax
import jax.numpy as jnp
from jax import lax
from jax.experimental import pallas as pl
from jax.experimental.pallas import tpu as pltpu
from jax.experimental.pallas import tpu_sc as plsc

NC = 2
NS = 16
CHUNK = 128
ROWS_PER_STEP = 8
GCOLS = 16


def _make_agg_kernel(n_nodes, idx_rows, acc_rows):
    rows_per_sub = idx_rows // NS
    steps = rows_per_sub // ROWS_PER_STEP
    half_rows_per_sub = idx_rows // 2 // NS
    half_steps = half_rows_per_sub // ROWS_PER_STEP
    zrows = acc_rows // NS
    mesh = plsc.VectorSubcoreMesh(core_axis_name="c", subcore_axis_name="s")
    oshape = jax.ShapeDtypeStruct((acc_rows, GCOLS), jnp.float32)

    @functools.partial(
        pl.kernel,
        out_type=(oshape, oshape, oshape, oshape),
        mesh=mesh,
        scratch_types=[
            pltpu.VMEM((2 * ROWS_PER_STEP, CHUNK), jnp.int32),
            pltpu.VMEM((2 * ROWS_PER_STEP, CHUNK), jnp.int32),
            pltpu.VMEM((ROWS_PER_STEP, CHUNK, GCOLS), jnp.float32),
            pltpu.VMEM_SHARED((acc_rows, GCOLS), jnp.float32),
            pltpu.SemaphoreType.DMA,
            pltpu.SemaphoreType.DMA,
            pltpu.SemaphoreType.DMA,
        ],
        compiler_params=pltpu.CompilerParams(use_tc_tiling_on_sc=False),
    )
    def agg_kernel(t0_hbm, t1_hbm, t2_hbm, idxb_hbm, zeros_hbm,
                   out0, out1, out2a, out2b,
                   ib0, ib1, rb, acc, gsem, ssem, isem):
        cid = lax.axis_index("c")
        sid = lax.axis_index("s")
        depth = 6
        blk = 2 * ROWS_PER_STEP

        def zero_acc():
            pltpu.sync_copy(zeros_hbm.at[pl.ds(sid * zrows, zrows)],
                            acc.at[pl.ds(sid * zrows, zrows)])

        def scan(table, step_base, nsteps):
            pltpu.async_copy(idxb_hbm.at[pl.ds(step_base * blk, blk)],
                             ib0, isem)

            def run_step(b, cur, nxt, prefetch):
                pltpu.make_async_copy(
                    idxb_hbm.at[pl.ds(0, blk)], cur, isem).wait()
                if prefetch:
                    @pl.when(b + 1 < nsteps)
                    def _():
                        pltpu.async_copy(
                            idxb_hbm.at[pl.ds((step_base + b + 1) * blk, blk)],
                            nxt, isem)
                g = [pltpu.async_copy(table.at[cur.at[j]], rb.at[j], gsem)
                     for j in range(depth)]
                s = []
                for j in range(ROWS_PER_STEP):
                    g[j].wait()
                    if j + depth < ROWS_PER_STEP:
                        g.append(pltpu.async_copy(
                            table.at[cur.at[j + depth]], rb.at[j + depth],
                            gsem))
                    s.append(pltpu.async_copy(
                        rb.at[j], acc.at[cur.at[ROWS_PER_STEP + j]], ssem,
                        add=True))
                for d in s:
                    d.wait()

            def double_step(b2, carry):
                run_step(2 * b2, ib0, ib1, True)
                run_step(2 * b2 + 1, ib1, ib0, True)
                return carry
            lax.fori_loop(0, nsteps // 2, double_step, 0)
            if nsteps % 2:
                run_step(nsteps - 1, ib0, ib1, False)

        def dump(out):
            pltpu.sync_copy(acc.at[pl.ds(sid * zrows, zrows)],
                            out.at[pl.ds(sid * zrows, zrows)])

        zero_acc()
        plsc.subcore_barrier()

        @pl.when(cid == 0)
        def _():
            scan(t0_hbm, sid * steps, steps)

        @pl.when(cid == 1)
        def _():
            scan(t1_hbm, sid * steps, steps)
        plsc.subcore_barrier()

        @pl.when(cid == 0)
        def _():
            dump(out0)

        @pl.when(cid == 1)
        def _():
            dump(out1)
        plsc.subcore_barrier()

        zero_acc()
        plsc.subcore_barrier()
        scan(t2_hbm,
             cid * (idx_rows // 2 // ROWS_PER_STEP) + sid * half_steps,
             half_steps)
        plsc.subcore_barrier()

        @pl.when(cid == 0)
        def _():
            dump(out2a)

        @pl.when(cid == 1)
        def _():
            dump(out2b)

    return agg_kernel


def _dense_body(x_ref, h_ref, c0_ref, a0_ref, a1_ref, a2a_ref, a2b_ref,
                w_ref, b_ref, lw_ref, lb_ref,
                out_ref, hn_ref, c_ref, *, filt, lags, kdim):
    def dot(a, b):
        return lax.dot_general(a, b, (((1,), (0,)), ((), ())),
                               precision=lax.Precision.HIGHEST,
                               preferred_element_type=jnp.float32)
    g4 = 4 * filt
    a2a = a2a_ref[...]
    a2b = a2b_ref[...]
    zc = jnp.concatenate(
        [x_ref[...], h_ref[...], a0_ref[...], a1_ref[...], a2a, a2b], axis=1)
    o = dot(zc, w_ref[...])
    cnt = a2a[:, lags:lags + 1] + a2b[:, lags:lags + 1]
    inv = 1.0 / jnp.maximum(cnt, 1.0)
    pre = o[:, :g4] + inv * o[:, g4:2 * g4] + b_ref[...]
    i_g = jax.nn.sigmoid(pre[:, 0 * filt:1 * filt])
    f_g = jax.nn.sigmoid(pre[:, 1 * filt:2 * filt])
    t_g = jnp.tanh(pre[:, 2 * filt:3 * filt])
    o_g = jax.nn.sigmoid(pre[:, 3 * filt:4 * filt])
    c = f_g * c0_ref[...] + i_g * t_g
    hn = o_g * jnp.tanh(c)
    h = jnp.maximum(hn, 0.0)
    out_ref[...] = dot(h, lw_ref[...]) + lb_ref[...]
    hn_ref[...] = hn
    c_ref[...] = c


def _make_dense_kernel(n_nodes, lags, filt, kdim, row_block):
    grid = (n_nodes // row_block,)
    g4 = 4 * filt
    row = lambda i: (i, 0)
    rep = lambda i: (0, 0)
    return pl.pallas_call(
        functools.partial(_dense_body, filt=filt, lags=lags, kdim=kdim),
        grid=grid,
        in_specs=[
            pl.BlockSpec((row_block, lags), row),
            pl.BlockSpec((row_block, filt), row),
            pl.BlockSpec((row_block, filt), row),
            pl.BlockSpec((row_block, GCOLS), row),
            pl.BlockSpec((row_block, GCOLS), row),
            pl.BlockSpec((row_block, GCOLS), row),
            pl.BlockSpec((row_block, GCOLS), row),
            pl.BlockSpec((kdim, 2 * g4), rep),
            pl.BlockSpec((1, g4), rep),
            pl.BlockSpec((filt, 1), rep),
            pl.BlockSpec((1, 1), rep),
        ],
        out_specs=[
            pl.BlockSpec((row_block, 1), row),
            pl.BlockSpec((row_block, filt), row),
            pl.BlockSpec((row_block, filt), row),
        ],
        out_shape=[
            jax.ShapeDtypeStruct((n_nodes, 1), jnp.float32),
            jax.ShapeDtypeStruct((n_nodes, filt), jnp.float32),
            jax.ShapeDtypeStruct((n_nodes, filt), jnp.float32),
        ],
    )


def kernel(x, edge_index, edge_weight, h_0, c_0, params):
    n, lags = x.shape
    filt = h_0.shape[1]
    e = edge_index.shape[1]

    t0 = h_0[:, :GCOLS]
    t1 = h_0[:, GCOLS:2 * GCOLS]
    t2 = jnp.concatenate(
        [x, jnp.ones((n, 1), jnp.float32),
         jnp.zeros((n, GCOLS - lags - 1), jnp.float32)], axis=1)

    rows = -(-e // CHUNK)
    blk = 2 * NS * ROWS_PER_STEP
    rows_p = -(-rows // blk) * blk
    e_pad = rows_p * CHUNK - e
    acc_rows = ((n + 1 + 8 * NS - 1) // (8 * NS)) * (8 * NS)
    pad_src = (jnp.arange(e_pad, dtype=jnp.int32) * 37) % n
    pad_dst = n + (jnp.arange(e_pad, dtype=jnp.int32) % (acc_rows - n))
    src2 = jnp.concatenate(
        [edge_index[0].astype(jnp.int32), pad_src]).reshape(
            rows_p // ROWS_PER_STEP, ROWS_PER_STEP, CHUNK)
    dst2 = jnp.concatenate(
        [edge_index[1].astype(jnp.int32), pad_dst]).reshape(
            rows_p // ROWS_PER_STEP, ROWS_PER_STEP, CHUNK)
    idxb = jnp.concatenate([src2, dst2], axis=1).reshape(2 * rows_p, CHUNK)

    zeros_hbm = jnp.zeros((acc_rows, GCOLS), jnp.float32)

    a0, a1, a2a, a2b = _make_agg_kernel(n, rows_p, acc_rows)(
        t0, t1, t2, idxb, zeros_hbm)

    gates_x = ['x_i', 'x_f', 'x_c', 'x_o']
    gates_h = ['h_i', 'h_f', 'h_c', 'h_o']

    def rel_w(p):
        return jnp.einsum('rb,bio->rio', p['comp'], p['basis'])[0]

    g4 = 4 * filt
    wx = jnp.concatenate([params[g]['root'] for g in gates_x], axis=1)
    wh = jnp.concatenate([params[g]['root'] for g in gates_h], axis=1)
    wax = jnp.concatenate([rel_w(params[g]) for g in gates_x], axis=1)
    wah = jnp.concatenate([rel_w(params[g]) for g in gates_h], axis=1)
    bias = jnp.concatenate(
        [params[gx]['bias'] + params[gh]['bias']
         for gx, gh in zip(gates_x, gates_h)])[None, :]
    w2 = jnp.concatenate(
        [wax, jnp.zeros((GCOLS - lags, g4), jnp.float32)], axis=0)

    kdim = lags + filt + 4 * GCOLS
    w_left = jnp.concatenate([wx, wh, jnp.zeros((4 * GCOLS, g4))], axis=0)
    w_right = jnp.concatenate(
        [jnp.zeros((lags + filt, g4)), wah[:GCOLS], wah[GCOLS:2 * GCOLS],
         w2, w2], axis=0)
    w256 = jnp.concatenate([w_left, w_right], axis=1)

    row_block = 2000
    out, h_new, c = _make_dense_kernel(n, lags, filt, kdim, row_block)(
        x, h_0, c_0, a0, a1, a2a, a2b,
        w256, bias, params['lin_w'], params['lin_b'].reshape(1, 1))
    return (out, h_new, c)
